# Initial kernel scaffold; baseline (speedup 1.0000x reference)
import jax
import jax.numpy as jnp
from jax.experimental import pallas as pl


def kernel(x, edge_index, W, b, Wg, bg):
    raise NotImplementedError("write your pallas kernel here")


# speedup vs baseline: 3.8129x; 3.8129x over previous
import functools

import jax
import jax.numpy as jnp
from jax import lax
from jax.experimental import pallas as pl
from jax.experimental.pallas import tpu as pltpu
from jax.experimental.pallas import tpu_sc as plsc

N_NODES = 10000
N_EDGES = 320000
D = 128

NC = 2
NS = 16
E_PER_TILE = N_EDGES // NS
CHUNK = 80
N_CHUNKS = E_PER_TILE // CHUNK
N_PAD = 10240
ROWS_PER_TILE = N_PAD // NS


@functools.partial(
    pl.kernel,
    mesh=plsc.VectorSubcoreMesh(core_axis_name="c", subcore_axis_name="s"),
    out_type=(
        jax.ShapeDtypeStruct((N_PAD, D), jnp.float32),
        jax.ShapeDtypeStruct((N_PAD, D), jnp.float32),
    ),
    scratch_types=[
        pltpu.VMEM_SHARED((N_PAD, D), jnp.float32),
        pltpu.VMEM((CHUNK,), jnp.int32),
        pltpu.VMEM((CHUNK,), jnp.int32),
        pltpu.VMEM((CHUNK, D), jnp.float32),
        pltpu.SemaphoreType.DMA,
    ],
)
def _sc_edge_agg(x_hbm, src_hbm, dst_hbm, zrows_hbm, ones_hbm,
                 agg_out, deg_out,
                 acc_sh, src_v, dst_v, rows_v, sem):
    c = lax.axis_index("c")
    s = lax.axis_index("s")

    row0 = s * ROWS_PER_TILE
    pltpu.sync_copy(zrows_hbm, acc_sh.at[pl.ds(row0, ROWS_PER_TILE)])
    plsc.subcore_barrier()

    base = s * E_PER_TILE

    @pl.when(c == 0)
    def _agg_phase():
        def chunk_body(i, carry):
            off = base + i * CHUNK
            pltpu.sync_copy(src_hbm.at[pl.ds(off, CHUNK)], src_v)
            pltpu.sync_copy(dst_hbm.at[pl.ds(off, CHUNK)], dst_v)
            pltpu.async_copy(x_hbm.at[src_v], rows_v, sem).wait()
            pltpu.sync_copy(rows_v, acc_sh.at[dst_v], add=True)
            return carry

        lax.fori_loop(0, N_CHUNKS, chunk_body, 0)

    @pl.when(c == 1)
    def _deg_phase():
        pltpu.sync_copy(ones_hbm, rows_v)

        def chunk_body(i, carry):
            off = base + i * CHUNK
            pltpu.sync_copy(dst_hbm.at[pl.ds(off, CHUNK)], dst_v)
            pltpu.sync_copy(rows_v, acc_sh.at[dst_v], add=True)
            return carry

        lax.fori_loop(0, N_CHUNKS, chunk_body, 0)

    plsc.subcore_barrier()

    @pl.when(c == 0)
    def _pub_agg():
        pltpu.sync_copy(acc_sh.at[pl.ds(row0, ROWS_PER_TILE)],
                        agg_out.at[pl.ds(row0, ROWS_PER_TILE)])

    @pl.when(c == 1)
    def _pub_deg():
        pltpu.sync_copy(acc_sh.at[pl.ds(row0, ROWS_PER_TILE)],
                        deg_out.at[pl.ds(row0, ROWS_PER_TILE)])


ROW_BLK = 1024


def _tc_finish_body(x_ref, agg_ref, deg_ref, W_ref, Wg_ref, b_ref, bg_ref,
                    out_ref):
    x = x_ref[...]
    deg = deg_ref[:, 0:1]
    agg = agg_ref[...] / jnp.clip(deg, 1.0, None)
    W = W_ref[...]
    Wg = Wg_ref[...]
    h = jnp.tanh(
        jnp.dot(x, W[:D], preferred_element_type=jnp.float32)
        + jnp.dot(agg, W[D:], preferred_element_type=jnp.float32)
        + b_ref[...])
    g = jax.nn.sigmoid(
        jnp.dot(x, Wg[:D], preferred_element_type=jnp.float32)
        + jnp.dot(agg, Wg[D:], preferred_element_type=jnp.float32)
        + bg_ref[...])
    row = lax.broadcasted_iota(jnp.int32, (D, D), 0)
    col = lax.broadcasted_iota(jnp.int32, (D, D), 1)
    T = jnp.where(row <= col, 1.0, 0.0) / (col.astype(jnp.float32) + 1.0)
    gate = jnp.dot(g, T, preferred_element_type=jnp.float32)
    out_ref[...] = gate * x + (1.0 - gate) * h


_tc_finish = pl.pallas_call(
    _tc_finish_body,
    grid=(N_PAD // ROW_BLK,),
    in_specs=[
        pl.BlockSpec((ROW_BLK, D), lambda i: (i, 0)),
        pl.BlockSpec((ROW_BLK, D), lambda i: (i, 0)),
        pl.BlockSpec((ROW_BLK, D), lambda i: (i, 0)),
        pl.BlockSpec((2 * D, D), lambda i: (0, 0)),
        pl.BlockSpec((2 * D, D), lambda i: (0, 0)),
        pl.BlockSpec((1, D), lambda i: (0, 0)),
        pl.BlockSpec((1, D), lambda i: (0, 0)),
    ],
    out_specs=pl.BlockSpec((ROW_BLK, D), lambda i: (i, 0)),
    out_shape=jax.ShapeDtypeStruct((N_PAD, D), jnp.float32),
)


def kernel(x, edge_index, W, b, Wg, bg):
    ei = edge_index.astype(jnp.int32)
    src = ei[0]
    dst = ei[1]
    zrows = jnp.zeros((ROWS_PER_TILE, D), jnp.float32)
    ones_rows = jnp.ones((CHUNK, D), jnp.float32)
    agg_sum, deg = _sc_edge_agg(x, src, dst, zrows, ones_rows)
    x_pad = jnp.concatenate(
        [x, jnp.zeros((N_PAD - N_NODES, D), jnp.float32)], axis=0)
    out = _tc_finish(x_pad, agg_sum, deg, W, Wg,
                     b.reshape(1, D), bg.reshape(1, D))
    return out[:N_NODES]


# speedup vs baseline: 4.0246x; 1.0555x over previous
import functools

import jax
import jax.numpy as jnp
from jax import lax
from jax.experimental import pallas as pl
from jax.experimental.pallas import tpu as pltpu
from jax.experimental.pallas import tpu_sc as plsc

N_NODES = 10000
N_EDGES = 320000
D = 128

NC = 2
NS = 16
CHUNK = 128
N_CHUNKS = 160
E_PER_TILE = N_CHUNKS * CHUNK
E_PAD = NS * E_PER_TILE
N_PAD = 10240
ROWS_PER_TILE = N_PAD // NS
BLK = 32
N_BLKS = N_CHUNKS // BLK
DEG_K = 8


@functools.partial(
    pl.kernel,
    mesh=plsc.VectorSubcoreMesh(core_axis_name="c", subcore_axis_name="s"),
    out_type=(
        jax.ShapeDtypeStruct((N_PAD, D), jnp.float32),
        jax.ShapeDtypeStruct((N_PAD, D), jnp.float32),
    ),
    scratch_types=[
        pltpu.VMEM_SHARED((N_PAD, D), jnp.float32),
        pltpu.VMEM((BLK, CHUNK), jnp.int32),
        pltpu.VMEM((BLK, CHUNK), jnp.int32),
        pltpu.VMEM((CHUNK, D), jnp.float32),
        pltpu.VMEM((CHUNK, D), jnp.float32),
        pltpu.SemaphoreType.DMA,
        pltpu.SemaphoreType.DMA,
    ],
)
def _sc_edge_agg(x_hbm, src_hbm, dst_hbm, zrows_hbm, ones_hbm,
                 agg_out, deg_out,
                 acc_sh, src_a, dst_a, rows0, rows1, sem0, sem1):
    c = lax.axis_index("c")
    s = lax.axis_index("s")

    row0 = s * ROWS_PER_TILE
    pltpu.sync_copy(zrows_hbm, acc_sh.at[pl.ds(row0, ROWS_PER_TILE)])
    plsc.subcore_barrier()

    @pl.when(c == 0)
    def _agg_phase():
        def fire(j, rows, sem):
            pltpu.async_copy(x_hbm.at[src_a.at[j]], rows, sem)

        def wait(rows, sem):
            pltpu.make_async_copy(x_hbm.at[src_a.at[0]], rows, sem).wait()

        def scat(j, rows):
            pltpu.sync_copy(rows, acc_sh.at[dst_a.at[j]], add=True)

        def blk_body(blk, carry):
            c0 = blk * BLK
            pltpu.sync_copy(src_hbm.at[s, pl.ds(c0, BLK)], src_a)
            pltpu.sync_copy(dst_hbm.at[s, pl.ds(c0, BLK)], dst_a)
            fire(0, rows0, sem0)

            def pair(p, carry2):
                j = 2 * p
                fire(j + 1, rows1, sem1)
                wait(rows0, sem0)
                scat(j, rows0)
                fire(j + 2, rows0, sem0)
                wait(rows1, sem1)
                scat(j + 1, rows1)
                return carry2

            lax.fori_loop(0, BLK // 2 - 1, pair, 0)
            fire(BLK - 1, rows1, sem1)
            wait(rows0, sem0)
            scat(BLK - 2, rows0)
            wait(rows1, sem1)
            scat(BLK - 1, rows1)
            return carry

        lax.fori_loop(0, N_BLKS, blk_body, 0)

    @pl.when(c == 1)
    def _deg_phase():
        pltpu.sync_copy(ones_hbm, rows0)

        def blk_body(blk, carry):
            c0 = blk * BLK
            pltpu.sync_copy(dst_hbm.at[s, pl.ds(c0, BLK)], dst_a)

            def group(gi, carry2):
                j0 = gi * DEG_K
                for k in range(DEG_K):
                    pltpu.async_copy(rows0, acc_sh.at[dst_a.at[j0 + k]],
                                     sem0, add=True)
                for k in range(DEG_K):
                    pltpu.make_async_copy(rows0, acc_sh.at[dst_a.at[0]],
                                          sem0).wait()
                return carry2

            lax.fori_loop(0, BLK // DEG_K, group, 0)
            return carry

        lax.fori_loop(0, N_BLKS, blk_body, 0)

    plsc.subcore_barrier()

    @pl.when(c == 0)
    def _pub_agg():
        pltpu.sync_copy(acc_sh.at[pl.ds(row0, ROWS_PER_TILE)],
                        agg_out.at[pl.ds(row0, ROWS_PER_TILE)])

    @pl.when(c == 1)
    def _pub_deg():
        pltpu.sync_copy(acc_sh.at[pl.ds(row0, ROWS_PER_TILE)],
                        deg_out.at[pl.ds(row0, ROWS_PER_TILE)])


ROW_BLK = 1024


def _tc_finish_body(x_ref, agg_ref, deg_ref, W_ref, Wg_ref, b_ref, bg_ref,
                    out_ref):
    x = x_ref[...]
    deg = deg_ref[:, 0:1]
    agg = agg_ref[...] / jnp.clip(deg, 1.0, None)
    W = W_ref[...]
    Wg = Wg_ref[...]
    h = jnp.tanh(
        jnp.dot(x, W[:D], preferred_element_type=jnp.float32)
        + jnp.dot(agg, W[D:], preferred_element_type=jnp.float32)
        + b_ref[...])
    g = jax.nn.sigmoid(
        jnp.dot(x, Wg[:D], preferred_element_type=jnp.float32)
        + jnp.dot(agg, Wg[D:], preferred_element_type=jnp.float32)
        + bg_ref[...])
    row = lax.broadcasted_iota(jnp.int32, (D, D), 0)
    col = lax.broadcasted_iota(jnp.int32, (D, D), 1)
    T = jnp.where(row <= col, 1.0, 0.0) / (col.astype(jnp.float32) + 1.0)
    gate = jnp.dot(g, T, preferred_element_type=jnp.float32)
    out_ref[...] = gate * x + (1.0 - gate) * h


_tc_finish = pl.pallas_call(
    _tc_finish_body,
    grid=(N_PAD // ROW_BLK,),
    in_specs=[
        pl.BlockSpec((ROW_BLK, D), lambda i: (i, 0)),
        pl.BlockSpec((ROW_BLK, D), lambda i: (i, 0)),
        pl.BlockSpec((ROW_BLK, D), lambda i: (i, 0)),
        pl.BlockSpec((2 * D, D), lambda i: (0, 0)),
        pl.BlockSpec((2 * D, D), lambda i: (0, 0)),
        pl.BlockSpec((1, D), lambda i: (0, 0)),
        pl.BlockSpec((1, D), lambda i: (0, 0)),
    ],
    out_specs=pl.BlockSpec((ROW_BLK, D), lambda i: (i, 0)),
    out_shape=jax.ShapeDtypeStruct((N_PAD, D), jnp.float32),
)


def kernel(x, edge_index, W, b, Wg, bg):
    ei = edge_index.astype(jnp.int32)
    npad = E_PAD - N_EDGES
    src = jnp.concatenate(
        [ei[0], jnp.zeros((npad,), jnp.int32)]).reshape(NS, N_CHUNKS, CHUNK)
    dst = jnp.concatenate(
        [ei[1], jnp.full((npad,), N_NODES, jnp.int32)]).reshape(
            NS, N_CHUNKS, CHUNK)
    zrows = jnp.zeros((ROWS_PER_TILE, D), jnp.float32)
    ones_rows = jnp.ones((CHUNK, D), jnp.float32)
    agg_sum, deg = _sc_edge_agg(x, src, dst, zrows, ones_rows)
    x_pad = jnp.concatenate(
        [x, jnp.zeros((N_PAD - N_NODES, D), jnp.float32)], axis=0)
    out = _tc_finish(x_pad, agg_sum, deg, W, Wg,
                     b.reshape(1, D), bg.reshape(1, D))
    return out[:N_NODES]


# speedup vs baseline: 4.2060x; 1.0451x over previous
import functools

import jax
import jax.numpy as jnp
from jax import lax
from jax.experimental import pallas as pl
from jax.experimental.pallas import tpu as pltpu
from jax.experimental.pallas import tpu_sc as plsc

N_NODES = 10000
N_EDGES = 320000
D = 128

NC = 2
NS = 16
CHUNK = 128
N_CHUNKS = 80
E_PER_TILE = N_CHUNKS * CHUNK
E_PAD = NC * NS * E_PER_TILE
N_PAD = 10240
ROWS_PER_TILE = N_PAD // NS
BLK = 16
N_BLKS = N_CHUNKS // BLK
DEG_K = 8


@functools.partial(
    pl.kernel,
    mesh=plsc.VectorSubcoreMesh(core_axis_name="c", subcore_axis_name="s"),
    out_type=(
        jax.ShapeDtypeStruct((NC, N_PAD, D), jnp.float32),
        jax.ShapeDtypeStruct((NC, N_PAD, D), jnp.float32),
    ),
    scratch_types=[
        pltpu.VMEM_SHARED((N_PAD, D), jnp.float32),
        pltpu.VMEM((BLK, CHUNK), jnp.int32),
        pltpu.VMEM((BLK, CHUNK), jnp.int32),
        pltpu.VMEM((CHUNK, D), jnp.float32),
        pltpu.VMEM((CHUNK, D), jnp.float32),
        pltpu.SemaphoreType.DMA,
        pltpu.SemaphoreType.DMA,
    ],
)
def _sc_edge_agg(x_hbm, src_hbm, dst_hbm, zrows_hbm, ones_hbm,
                 agg_out, deg_out,
                 acc_sh, src_a, dst_a, rows0, rows1, sem0, sem1):
    c = lax.axis_index("c")
    s = lax.axis_index("s")

    row0 = s * ROWS_PER_TILE
    pltpu.sync_copy(zrows_hbm, acc_sh.at[pl.ds(row0, ROWS_PER_TILE)])
    plsc.subcore_barrier()

    def fire(j, rows, sem):
        pltpu.async_copy(x_hbm.at[src_a.at[j]], rows, sem)

    def wait(rows, sem):
        pltpu.make_async_copy(x_hbm.at[src_a.at[0]], rows, sem).wait()

    def scat(j, rows):
        pltpu.sync_copy(rows, acc_sh.at[dst_a.at[j]], add=True)

    def agg_blk(blk, carry):
        c0 = blk * BLK
        pltpu.sync_copy(src_hbm.at[c, s, pl.ds(c0, BLK)], src_a)
        pltpu.sync_copy(dst_hbm.at[c, s, pl.ds(c0, BLK)], dst_a)
        fire(0, rows0, sem0)

        def pair(p, carry2):
            j = 2 * p
            fire(j + 1, rows1, sem1)
            wait(rows0, sem0)
            scat(j, rows0)
            fire(j + 2, rows0, sem0)
            wait(rows1, sem1)
            scat(j + 1, rows1)
            return carry2

        lax.fori_loop(0, BLK // 2 - 1, pair, 0)
        fire(BLK - 1, rows1, sem1)
        wait(rows0, sem0)
        scat(BLK - 2, rows0)
        wait(rows1, sem1)
        scat(BLK - 1, rows1)
        return carry

    lax.fori_loop(0, N_BLKS, agg_blk, 0)
    plsc.subcore_barrier()

    pltpu.sync_copy(acc_sh.at[pl.ds(row0, ROWS_PER_TILE)],
                    agg_out.at[c, pl.ds(row0, ROWS_PER_TILE)])
    pltpu.sync_copy(zrows_hbm, acc_sh.at[pl.ds(row0, ROWS_PER_TILE)])
    plsc.subcore_barrier()

    pltpu.sync_copy(ones_hbm, rows0)

    def deg_blk(blk, carry):
        c0 = blk * BLK
        pltpu.sync_copy(dst_hbm.at[c, s, pl.ds(c0, BLK)], dst_a)

        def group(gi, carry2):
            j0 = gi * DEG_K
            for k in range(DEG_K):
                pltpu.async_copy(rows0, acc_sh.at[dst_a.at[j0 + k]],
                                 sem0, add=True)
            for k in range(DEG_K):
                pltpu.make_async_copy(rows0, acc_sh.at[dst_a.at[0]],
                                      sem0).wait()
            return carry2

        lax.fori_loop(0, BLK // DEG_K, group, 0)
        return carry

    lax.fori_loop(0, N_BLKS, deg_blk, 0)
    plsc.subcore_barrier()

    pltpu.sync_copy(acc_sh.at[pl.ds(row0, ROWS_PER_TILE)],
                    deg_out.at[c, pl.ds(row0, ROWS_PER_TILE)])


ROW_BLK = 1024


def _tc_finish_body(x_ref, agg_ref, deg_ref, W_ref, Wg_ref, b_ref, bg_ref,
                    out_ref):
    x = x_ref[...]
    deg = deg_ref[0, :, 0:1] + deg_ref[1, :, 0:1]
    agg = (agg_ref[0] + agg_ref[1]) / jnp.clip(deg, 1.0, None)
    W = W_ref[...]
    Wg = Wg_ref[...]
    h = jnp.tanh(
        jnp.dot(x, W[:D], preferred_element_type=jnp.float32)
        + jnp.dot(agg, W[D:], preferred_element_type=jnp.float32)
        + b_ref[...])
    g = jax.nn.sigmoid(
        jnp.dot(x, Wg[:D], preferred_element_type=jnp.float32)
        + jnp.dot(agg, Wg[D:], preferred_element_type=jnp.float32)
        + bg_ref[...])
    row = lax.broadcasted_iota(jnp.int32, (D, D), 0)
    col = lax.broadcasted_iota(jnp.int32, (D, D), 1)
    T = jnp.where(row <= col, 1.0, 0.0) / (col.astype(jnp.float32) + 1.0)
    gate = jnp.dot(g, T, preferred_element_type=jnp.float32)
    out_ref[...] = gate * x + (1.0 - gate) * h


_tc_finish = pl.pallas_call(
    _tc_finish_body,
    grid=(N_PAD // ROW_BLK,),
    in_specs=[
        pl.BlockSpec((ROW_BLK, D), lambda i: (i, 0)),
        pl.BlockSpec((NC, ROW_BLK, D), lambda i: (0, i, 0)),
        pl.BlockSpec((NC, ROW_BLK, D), lambda i: (0, i, 0)),
        pl.BlockSpec((2 * D, D), lambda i: (0, 0)),
        pl.BlockSpec((2 * D, D), lambda i: (0, 0)),
        pl.BlockSpec((1, D), lambda i: (0, 0)),
        pl.BlockSpec((1, D), lambda i: (0, 0)),
    ],
    out_specs=pl.BlockSpec((ROW_BLK, D), lambda i: (i, 0)),
    out_shape=jax.ShapeDtypeStruct((N_PAD, D), jnp.float32),
)


def kernel(x, edge_index, W, b, Wg, bg):
    ei = edge_index.astype(jnp.int32)
    npad = E_PAD - N_EDGES
    src = jnp.concatenate(
        [ei[0], jnp.zeros((npad,), jnp.int32)]).reshape(
            NC, NS, N_CHUNKS, CHUNK)
    dst_pad = N_NODES + jnp.arange(npad, dtype=jnp.int32) % (N_PAD - N_NODES)
    dst = jnp.concatenate([ei[1], dst_pad]).reshape(NC, NS, N_CHUNKS, CHUNK)
    zrows = jnp.zeros((ROWS_PER_TILE, D), jnp.float32)
    ones_rows = jnp.ones((CHUNK, D), jnp.float32)
    agg_part, deg_part = _sc_edge_agg(x, src, dst, zrows, ones_rows)
    x_pad = jnp.concatenate(
        [x, jnp.zeros((N_PAD - N_NODES, D), jnp.float32)], axis=0)
    out = _tc_finish(x_pad, agg_part, deg_part, W, Wg,
                     b.reshape(1, D), bg.reshape(1, D))
    return out[:N_NODES]


# speedup vs baseline: 10.2314x; 2.4326x over previous
import functools

import jax
import jax.numpy as jnp
from jax import lax
from jax.experimental import pallas as pl
from jax.experimental.pallas import tpu as pltpu
from jax.experimental.pallas import tpu_sc as plsc

N_NODES = 10000
N_EDGES = 320000
D = 128

NC = 2
NS = 16
CHUNK = 128
N_CHUNKS = 80
E_PER_TILE = N_CHUNKS * CHUNK
E_PAD = NC * NS * E_PER_TILE
N_PAD = 10240
ROWS_PER_TILE = N_PAD // NS
BLK = 16
N_BLKS = N_CHUNKS // BLK
DEG_K = 8


@functools.partial(
    pl.kernel,
    mesh=plsc.VectorSubcoreMesh(core_axis_name="c", subcore_axis_name="s"),
    out_type=(
        jax.ShapeDtypeStruct((NC, N_PAD, D), jnp.float32),
        jax.ShapeDtypeStruct((NC, N_PAD, D), jnp.float32),
    ),
    scratch_types=[
        pltpu.VMEM_SHARED((N_PAD, D), jnp.float32),
        pltpu.VMEM((BLK, CHUNK), jnp.int32),
        pltpu.VMEM((BLK, CHUNK), jnp.int32),
        pltpu.VMEM((CHUNK, D), jnp.float32),
        pltpu.VMEM((CHUNK, D), jnp.float32),
        pltpu.SemaphoreType.DMA,
        pltpu.SemaphoreType.DMA,
    ],
)
def _sc_edge_agg(x_hbm, src_hbm, dst_hbm, zrows_hbm, ones_hbm,
                 agg_out, deg_out,
                 acc_sh, src_a, dst_a, rows0, rows1, sem0, sem1):
    c = lax.axis_index("c")
    s = lax.axis_index("s")

    row0 = s * ROWS_PER_TILE
    pltpu.sync_copy(zrows_hbm, acc_sh.at[pl.ds(row0, ROWS_PER_TILE)])
    plsc.subcore_barrier()

    def fire(j, rows, sem):
        pltpu.async_copy(x_hbm.at[src_a.at[j]], rows, sem)

    def wait(rows, sem):
        pltpu.make_async_copy(x_hbm.at[src_a.at[0]], rows, sem).wait()

    def scat(j, rows):
        pltpu.sync_copy(rows, acc_sh.at[dst_a.at[j]], add=True)

    def agg_blk(blk, carry):
        c0 = blk * BLK
        pltpu.sync_copy(src_hbm.at[c, s, pl.ds(c0, BLK)], src_a)
        pltpu.sync_copy(dst_hbm.at[c, s, pl.ds(c0, BLK)], dst_a)
        fire(0, rows0, sem0)

        def pair(p, carry2):
            j = 2 * p
            fire(j + 1, rows1, sem1)
            wait(rows0, sem0)
            scat(j, rows0)
            fire(j + 2, rows0, sem0)
            wait(rows1, sem1)
            scat(j + 1, rows1)
            return carry2

        lax.fori_loop(0, BLK // 2 - 1, pair, 0)
        fire(BLK - 1, rows1, sem1)
        wait(rows0, sem0)
        scat(BLK - 2, rows0)
        wait(rows1, sem1)
        scat(BLK - 1, rows1)
        return carry

    lax.fori_loop(0, N_BLKS, agg_blk, 0)
    plsc.subcore_barrier()

    pltpu.sync_copy(acc_sh.at[pl.ds(row0, ROWS_PER_TILE)],
                    agg_out.at[c, pl.ds(row0, ROWS_PER_TILE)])
    pltpu.sync_copy(zrows_hbm, acc_sh.at[pl.ds(row0, ROWS_PER_TILE)])
    plsc.subcore_barrier()

    pltpu.sync_copy(ones_hbm, rows0)

    def deg_blk(blk, carry):
        c0 = blk * BLK
        pltpu.sync_copy(dst_hbm.at[c, s, pl.ds(c0, BLK)], dst_a)

        def group(gi, carry2):
            j0 = gi * DEG_K
            for k in range(DEG_K):
                pltpu.async_copy(rows0, acc_sh.at[dst_a.at[j0 + k]],
                                 sem0, add=True)
            for k in range(DEG_K):
                pltpu.make_async_copy(rows0, acc_sh.at[dst_a.at[0]],
                                      sem0).wait()
            return carry2

        lax.fori_loop(0, BLK // DEG_K, group, 0)
        return carry

    lax.fori_loop(0, N_BLKS, deg_blk, 0)
    plsc.subcore_barrier()

    pltpu.sync_copy(acc_sh.at[pl.ds(row0, ROWS_PER_TILE)],
                    deg_out.at[c, pl.ds(row0, ROWS_PER_TILE)])


ROW_BLK = 1000


def _tc_finish_body(x_ref, agg_ref, deg_ref, W_ref, Wg_ref, b_ref, bg_ref,
                    out_ref):
    x = x_ref[...]
    deg = deg_ref[0, :, 0:1] + deg_ref[1, :, 0:1]
    agg = (agg_ref[0] + agg_ref[1]) / jnp.clip(deg, 1.0, None)
    W = W_ref[...]
    Wg = Wg_ref[...]
    h = jnp.tanh(
        jnp.dot(x, W[:D], preferred_element_type=jnp.float32)
        + jnp.dot(agg, W[D:], preferred_element_type=jnp.float32)
        + b_ref[...])
    g = jax.nn.sigmoid(
        jnp.dot(x, Wg[:D], preferred_element_type=jnp.float32)
        + jnp.dot(agg, Wg[D:], preferred_element_type=jnp.float32)
        + bg_ref[...])
    row = lax.broadcasted_iota(jnp.int32, (D, D), 0)
    col = lax.broadcasted_iota(jnp.int32, (D, D), 1)
    T = jnp.where(row <= col, 1.0, 0.0) / (col.astype(jnp.float32) + 1.0)
    gate = jnp.dot(g, T, preferred_element_type=jnp.float32)
    out_ref[...] = gate * x + (1.0 - gate) * h


_tc_finish = pl.pallas_call(
    _tc_finish_body,
    grid=(N_NODES // ROW_BLK,),
    in_specs=[
        pl.BlockSpec((ROW_BLK, D), lambda i: (i, 0)),
        pl.BlockSpec((NC, ROW_BLK, D), lambda i: (0, i, 0)),
        pl.BlockSpec((NC, ROW_BLK, D), lambda i: (0, i, 0)),
        pl.BlockSpec((2 * D, D), lambda i: (0, 0)),
        pl.BlockSpec((2 * D, D), lambda i: (0, 0)),
        pl.BlockSpec((1, D), lambda i: (0, 0)),
        pl.BlockSpec((1, D), lambda i: (0, 0)),
    ],
    out_specs=pl.BlockSpec((ROW_BLK, D), lambda i: (i, 0)),
    out_shape=jax.ShapeDtypeStruct((N_NODES, D), jnp.float32),
)


def kernel(x, edge_index, W, b, Wg, bg):
    ei = edge_index.astype(jnp.int32)
    nt = NC * NS
    pad_per_tile = E_PER_TILE - N_EDGES // nt
    src_pad = jnp.broadcast_to(
        jnp.arange(pad_per_tile, dtype=jnp.int32) * 41 % N_NODES,
        (nt, pad_per_tile))
    dst_pad = jnp.broadcast_to(
        N_NODES + jnp.arange(pad_per_tile, dtype=jnp.int32),
        (nt, pad_per_tile))
    src = jnp.concatenate([ei[0].reshape(nt, -1), src_pad], axis=1).reshape(
        NC, NS, N_CHUNKS, CHUNK)
    dst = jnp.concatenate([ei[1].reshape(nt, -1), dst_pad], axis=1).reshape(
        NC, NS, N_CHUNKS, CHUNK)
    zrows = jnp.zeros((ROWS_PER_TILE, D), jnp.float32)
    ones_rows = jnp.ones((CHUNK, D), jnp.float32)
    agg_part, deg_part = _sc_edge_agg(x, src, dst, zrows, ones_rows)
    return _tc_finish(x, agg_part, deg_part, W, Wg,
                      b.reshape(1, D), bg.reshape(1, D))


# speedup vs baseline: 10.6490x; 1.0408x over previous
import functools

import jax
import jax.numpy as jnp
from jax import lax
from jax.experimental import pallas as pl
from jax.experimental.pallas import tpu as pltpu
from jax.experimental.pallas import tpu_sc as plsc

N_NODES = 10000
N_EDGES = 320000
D = 128

NC = 2
NS = 16
CHUNK = 128
N_CHUNKS = 80
E_PER_TILE = N_CHUNKS * CHUNK
E_PAD = NC * NS * E_PER_TILE
N_PAD = 10240
ROWS_PER_TILE = N_PAD // NS
BLK = 40
N_BLKS = N_CHUNKS // BLK
DEG_K = 8


@functools.partial(
    pl.kernel,
    mesh=plsc.VectorSubcoreMesh(core_axis_name="c", subcore_axis_name="s"),
    out_type=(
        jax.ShapeDtypeStruct((NC, N_PAD, D), jnp.float32),
        jax.ShapeDtypeStruct((NC, N_PAD, D), jnp.float32),
    ),
    scratch_types=[
        pltpu.VMEM_SHARED((N_PAD, D), jnp.float32),
        pltpu.VMEM((BLK, CHUNK), jnp.int32),
        pltpu.VMEM((BLK, CHUNK), jnp.int32),
        pltpu.VMEM((CHUNK, D), jnp.float32),
        pltpu.VMEM((CHUNK, D), jnp.float32),
        pltpu.SemaphoreType.DMA,
        pltpu.SemaphoreType.DMA,
    ],
)
def _sc_edge_agg(x_hbm, src_hbm, dst_hbm, zrows_hbm, ones_hbm,
                 agg_out, deg_out,
                 acc_sh, src_a, dst_a, rows0, rows1, sem0, sem1):
    c = lax.axis_index("c")
    s = lax.axis_index("s")

    row0 = s * ROWS_PER_TILE
    pltpu.sync_copy(zrows_hbm, acc_sh.at[pl.ds(row0, ROWS_PER_TILE)])
    plsc.subcore_barrier()

    def fire(j, rows, sem):
        pltpu.async_copy(x_hbm.at[src_a.at[j]], rows, sem)

    def wait(rows, sem):
        pltpu.make_async_copy(x_hbm.at[src_a.at[0]], rows, sem).wait()

    def scat(j, rows):
        pltpu.sync_copy(rows, acc_sh.at[dst_a.at[j]], add=True)

    def agg_blk(blk, carry):
        c0 = blk * BLK
        pltpu.sync_copy(src_hbm.at[c, s, pl.ds(c0, BLK)], src_a)
        pltpu.sync_copy(dst_hbm.at[c, s, pl.ds(c0, BLK)], dst_a)
        fire(0, rows0, sem0)

        def pair(p, carry2):
            j = 2 * p
            fire(j + 1, rows1, sem1)
            wait(rows0, sem0)
            scat(j, rows0)
            fire(j + 2, rows0, sem0)
            wait(rows1, sem1)
            scat(j + 1, rows1)
            return carry2

        lax.fori_loop(0, BLK // 2 - 1, pair, 0)
        fire(BLK - 1, rows1, sem1)
        wait(rows0, sem0)
        scat(BLK - 2, rows0)
        wait(rows1, sem1)
        scat(BLK - 1, rows1)
        return carry

    lax.fori_loop(0, N_BLKS, agg_blk, 0)
    plsc.subcore_barrier()

    pltpu.sync_copy(acc_sh.at[pl.ds(row0, ROWS_PER_TILE)],
                    agg_out.at[c, pl.ds(row0, ROWS_PER_TILE)])
    pltpu.sync_copy(zrows_hbm, acc_sh.at[pl.ds(row0, ROWS_PER_TILE)])
    plsc.subcore_barrier()

    pltpu.sync_copy(ones_hbm, rows0)

    def deg_blk(blk, carry):
        c0 = blk * BLK
        pltpu.sync_copy(dst_hbm.at[c, s, pl.ds(c0, BLK)], dst_a)

        def group(gi, carry2):
            j0 = gi * DEG_K
            for k in range(DEG_K):
                pltpu.async_copy(rows0, acc_sh.at[dst_a.at[j0 + k]],
                                 sem0, add=True)
            for k in range(DEG_K):
                pltpu.make_async_copy(rows0, acc_sh.at[dst_a.at[0]],
                                      sem0).wait()
            return carry2

        lax.fori_loop(0, BLK // DEG_K, group, 0)
        return carry

    lax.fori_loop(0, N_BLKS, deg_blk, 0)
    plsc.subcore_barrier()

    pltpu.sync_copy(acc_sh.at[pl.ds(row0, ROWS_PER_TILE)],
                    deg_out.at[c, pl.ds(row0, ROWS_PER_TILE)])


ROW_BLK = 1000


def _tc_finish_body(x_ref, agg_ref, deg_ref, W_ref, Wg_ref, b_ref, bg_ref,
                    out_ref):
    x = x_ref[...]
    deg = deg_ref[0] + deg_ref[1]
    agg = (agg_ref[0] + agg_ref[1]) / jnp.clip(deg, 1.0, None)
    W = W_ref[...]
    Wg = Wg_ref[...]
    h = jnp.tanh(
        jnp.dot(x, W[:D], preferred_element_type=jnp.float32)
        + jnp.dot(agg, W[D:], preferred_element_type=jnp.float32)
        + b_ref[...])
    g = jax.nn.sigmoid(
        jnp.dot(x, Wg[:D], preferred_element_type=jnp.float32)
        + jnp.dot(agg, Wg[D:], preferred_element_type=jnp.float32)
        + bg_ref[...])
    row = lax.broadcasted_iota(jnp.int32, (D, D), 0)
    col = lax.broadcasted_iota(jnp.int32, (D, D), 1)
    T = jnp.where(row <= col, 1.0, 0.0) / (col.astype(jnp.float32) + 1.0)
    gate = jnp.dot(g, T, preferred_element_type=jnp.float32)
    out_ref[...] = gate * x + (1.0 - gate) * h


_tc_finish = pl.pallas_call(
    _tc_finish_body,
    grid=(N_NODES // ROW_BLK,),
    in_specs=[
        pl.BlockSpec((ROW_BLK, D), lambda i: (i, 0)),
        pl.BlockSpec((NC, ROW_BLK, D), lambda i: (0, i, 0)),
        pl.BlockSpec((NC, ROW_BLK, 1), lambda i: (0, i, 0)),
        pl.BlockSpec((2 * D, D), lambda i: (0, 0)),
        pl.BlockSpec((2 * D, D), lambda i: (0, 0)),
        pl.BlockSpec((1, D), lambda i: (0, 0)),
        pl.BlockSpec((1, D), lambda i: (0, 0)),
    ],
    out_specs=pl.BlockSpec((ROW_BLK, D), lambda i: (i, 0)),
    out_shape=jax.ShapeDtypeStruct((N_NODES, D), jnp.float32),
)


def kernel(x, edge_index, W, b, Wg, bg):
    ei = edge_index.astype(jnp.int32)
    nt = NC * NS
    pad_per_tile = E_PER_TILE - N_EDGES // nt
    src_pad = jnp.broadcast_to(
        jnp.arange(pad_per_tile, dtype=jnp.int32) * 41 % N_NODES,
        (nt, pad_per_tile))
    dst_pad = jnp.broadcast_to(
        N_NODES + jnp.arange(pad_per_tile, dtype=jnp.int32),
        (nt, pad_per_tile))
    src = jnp.concatenate([ei[0].reshape(nt, -1), src_pad], axis=1).reshape(
        NC, NS, N_CHUNKS, CHUNK)
    dst = jnp.concatenate([ei[1].reshape(nt, -1), dst_pad], axis=1).reshape(
        NC, NS, N_CHUNKS, CHUNK)
    zrows = jnp.zeros((ROWS_PER_TILE, D), jnp.float32)
    ones_rows = jnp.ones((CHUNK, D), jnp.float32)
    agg_part, deg_part = _sc_edge_agg(x, src, dst, zrows, ones_rows)
    return _tc_finish(x, agg_part, deg_part[:, :, :1], W, Wg,
                      b.reshape(1, D), bg.reshape(1, D))
